# baseline (speedup 1.0000x reference)
import functools

import jax
import jax.numpy as jnp
from jax import lax
from jax.experimental import pallas as pl
from jax.experimental.pallas import tpu as pltpu
from jax.experimental.pallas import tpu_sc as plsc

N_NODES = 10000
D_EMBED = 256
D_HIDDEN = 512
N_EDGES = 160000

NC = 2
NS = 16
LANES = 16
HALF = D_EMBED // NC
CHUNK = 80
E_PER_TILE = N_EDGES // NS
NCHUNK = E_PER_TILE // CHUNK
N_PAD = 10240
ROWS_PER_TILE = N_PAD // NS


def _sc_body(src_hbm, dst_hbm, table_hbm, out_hbm,
             acc_sh, src_v, dst_v, data0, data1, gsem0, gsem1, ssem0, ssem1):
    c = lax.axis_index("c")
    s = lax.axis_index("s")

    pltpu.sync_copy(src_hbm.at[s], src_v)
    pltpu.sync_copy(dst_hbm.at[s], dst_v)

    @pl.loop(0, E_PER_TILE // LANES)
    def _xform(k):
        sl = pl.ds(k * LANES, LANES)
        src_v[sl] = src_v[sl] * 2 + c

    @pl.loop(0, CHUNK)
    def _zero(j):
        for k in range(HALF // LANES):
            data0[j, pl.ds(k * LANES, LANES)] = jnp.zeros((LANES,), jnp.float32)
    row0 = s * ROWS_PER_TILE
    for z in range(ROWS_PER_TILE // CHUNK):
        pltpu.sync_copy(data0, acc_sh.at[pl.ds(row0 + z * CHUNK, CHUNK)])
    plsc.subcore_barrier()

    bufs = ((data0, gsem0, ssem0), (data1, gsem1, ssem1))

    def _fire_gather(b, j):
        data, gsem, _ = bufs[b]
        pltpu.async_copy(table_hbm.at[src_v.at[pl.ds(j * CHUNK, CHUNK)]], data, gsem)

    def _wait_gather(b, j):
        data, gsem, _ = bufs[b]
        pltpu.make_async_copy(table_hbm.at[src_v.at[pl.ds(j * CHUNK, CHUNK)]], data, gsem).wait()

    def _fire_scatter(b, j):
        data, _, ssem = bufs[b]
        pltpu.async_copy(data, acc_sh.at[dst_v.at[j]], ssem, add=True)

    def _wait_scatter(b, j):
        data, _, ssem = bufs[b]
        pltpu.make_async_copy(data, acc_sh.at[dst_v.at[j]], ssem).wait()

    _fire_gather(0, 0)
    _fire_gather(1, 1)

    @pl.loop(0, NCHUNK, step=2)
    def _mainloop(jj):
        for b in range(2):
            j = jj + b

            @pl.when(j < NCHUNK)
            def _():
                _wait_gather(b, j)
                _fire_scatter(b, j)

                @pl.when(j + 2 < NCHUNK)
                def _():
                    _wait_scatter(b, j)
                    _fire_gather(b, j + 2)

    _wait_scatter(1, NCHUNK - 2)
    _wait_scatter(0, NCHUNK - 1)
    plsc.subcore_barrier()
    pltpu.sync_copy(acc_sh.at[pl.ds(row0, ROWS_PER_TILE)],
                    out_hbm.at[c, pl.ds(row0, ROWS_PER_TILE)])


def _sc_messages(src2d, dst2d, table2):
    f = pl.kernel(
        _sc_body,
        out_type=jax.ShapeDtypeStruct((NC, N_PAD, HALF), jnp.float32),
        mesh=plsc.VectorSubcoreMesh(core_axis_name="c", subcore_axis_name="s",
                                    num_cores=NC, num_subcores=NS),
        scratch_types=[
            pltpu.VMEM_SHARED((N_PAD, HALF), jnp.float32),
            pltpu.VMEM((E_PER_TILE,), jnp.int32),
            pltpu.VMEM((NCHUNK, CHUNK), jnp.int32),
            pltpu.VMEM((CHUNK, HALF), jnp.float32),
            pltpu.VMEM((CHUNK, HALF), jnp.float32),
            pltpu.SemaphoreType.DMA,
            pltpu.SemaphoreType.DMA,
            pltpu.SemaphoreType.DMA,
            pltpu.SemaphoreType.DMA,
        ],
    )
    return f(src2d, dst2d, table2)


BN = 1000


def _mlp_body(msg_ref, table_ref, w1_ref, b1_ref, w2_ref, b2_ref, out_ref):
    x0 = msg_ref[0]
    x1 = msg_ref[1]
    h = jnp.dot(x0, w1_ref[:HALF, :], preferred_element_type=jnp.float32)
    h = h + jnp.dot(x1, w1_ref[HALF:, :], preferred_element_type=jnp.float32)
    h = jnp.maximum(h + b1_ref[...], 0.0)
    u = jnp.dot(h, w2_ref[...], preferred_element_type=jnp.float32)
    u = jnp.maximum(u + b2_ref[...], 0.0)
    out_ref[...] = table_ref[...] + u


def _mlp(msg, table, W1, b1, W2, b2):
    return pl.pallas_call(
        _mlp_body,
        grid=(N_NODES // BN,),
        in_specs=[
            pl.BlockSpec((NC, BN, HALF), lambda i: (0, i, 0)),
            pl.BlockSpec((BN, D_EMBED), lambda i: (i, 0)),
            pl.BlockSpec((D_EMBED, D_HIDDEN), lambda i: (0, 0)),
            pl.BlockSpec((1, D_HIDDEN), lambda i: (0, 0)),
            pl.BlockSpec((D_HIDDEN, D_EMBED), lambda i: (0, 0)),
            pl.BlockSpec((1, D_EMBED), lambda i: (0, 0)),
        ],
        out_specs=pl.BlockSpec((BN, D_EMBED), lambda i: (i, 0)),
        out_shape=jax.ShapeDtypeStruct((N_NODES, D_EMBED), jnp.float32),
    )(msg, table, W1, b1, W2, b2)


def kernel(edge_index, table, W1, b1, W2, b2):
    src2d = edge_index[0].reshape(NS, E_PER_TILE)
    dst2d = edge_index[1].reshape(NS, NCHUNK, CHUNK)
    table2 = table.reshape(NC * N_NODES, HALF)
    msg = _sc_messages(src2d, dst2d, table2)
    return _mlp(msg, table, W1, b1.reshape(1, D_HIDDEN), W2, b2.reshape(1, D_EMBED))


# speedup vs baseline: 1.0231x; 1.0231x over previous
import functools

import jax
import jax.numpy as jnp
from jax import lax
from jax.experimental import pallas as pl
from jax.experimental.pallas import tpu as pltpu
from jax.experimental.pallas import tpu_sc as plsc

N_NODES = 10000
D_EMBED = 256
D_HIDDEN = 512
N_EDGES = 160000

NC = 2
NS = 16
LANES = 16
HALF = D_EMBED // NC
CHUNK = 80
E_PER_TILE = N_EDGES // NS
NCHUNK = E_PER_TILE // CHUNK
N_PAD = 10240
ROWS_PER_TILE = N_PAD // NS


def _sc_body(src_hbm, dst_hbm, table_hbm, out_hbm,
             acc_sh, src_v, dst_v, data0, data1, gsem0, gsem1, ssem0, ssem1,
             gsem0b, gsem1b):
    c = lax.axis_index("c")
    s = lax.axis_index("s")

    pltpu.sync_copy(src_hbm.at[s], src_v)
    pltpu.sync_copy(dst_hbm.at[s], dst_v)

    @pl.loop(0, E_PER_TILE // LANES)
    def _xform(k):
        sl = pl.ds(k * LANES, LANES)
        src_v[sl] = src_v[sl] * 2 + c

    @pl.loop(0, CHUNK)
    def _zero(j):
        for k in range(HALF // LANES):
            data0[j, pl.ds(k * LANES, LANES)] = jnp.zeros((LANES,), jnp.float32)
    row0 = s * ROWS_PER_TILE
    for z in range(ROWS_PER_TILE // CHUNK):
        pltpu.sync_copy(data0, acc_sh.at[pl.ds(row0 + z * CHUNK, CHUNK)])
    plsc.subcore_barrier()

    bufs = ((data0, gsem0, ssem0), (data1, gsem1, ssem1))

    H = CHUNK // 2
    bsems = (gsem0b, gsem1b)

    def _fire_gather(b, j):
        data, gsem, _ = bufs[b]
        pltpu.async_copy(table_hbm.at[src_v.at[pl.ds(j * CHUNK, H)]],
                         data.at[pl.ds(0, H)], gsem)
        pltpu.async_copy(table_hbm.at[src_v.at[pl.ds(j * CHUNK + H, H)]],
                         data.at[pl.ds(H, H)], bsems[b])

    def _wait_gather(b, j):
        data, gsem, _ = bufs[b]
        pltpu.make_async_copy(table_hbm.at[src_v.at[pl.ds(j * CHUNK, H)]],
                              data.at[pl.ds(0, H)], gsem).wait()
        pltpu.make_async_copy(table_hbm.at[src_v.at[pl.ds(j * CHUNK + H, H)]],
                              data.at[pl.ds(H, H)], bsems[b]).wait()

    def _fire_scatter(b, j):
        data, _, ssem = bufs[b]
        pltpu.async_copy(data, acc_sh.at[dst_v.at[j]], ssem, add=True)

    def _wait_scatter(b, j):
        data, _, ssem = bufs[b]
        pltpu.make_async_copy(data, acc_sh.at[dst_v.at[j]], ssem).wait()

    _fire_gather(0, 0)
    _fire_gather(1, 1)

    @pl.loop(0, NCHUNK, step=2)
    def _mainloop(jj):
        for b in range(2):
            j = jj + b

            @pl.when(j < NCHUNK)
            def _():
                _wait_gather(b, j)
                _fire_scatter(b, j)

                @pl.when(j + 2 < NCHUNK)
                def _():
                    _wait_scatter(b, j)
                    _fire_gather(b, j + 2)

    _wait_scatter(1, NCHUNK - 2)
    _wait_scatter(0, NCHUNK - 1)
    plsc.subcore_barrier()
    pltpu.sync_copy(acc_sh.at[pl.ds(row0, ROWS_PER_TILE)],
                    out_hbm.at[c, pl.ds(row0, ROWS_PER_TILE)])


def _sc_messages(src2d, dst2d, table2):
    f = pl.kernel(
        _sc_body,
        out_type=jax.ShapeDtypeStruct((NC, N_PAD, HALF), jnp.float32),
        mesh=plsc.VectorSubcoreMesh(core_axis_name="c", subcore_axis_name="s",
                                    num_cores=NC, num_subcores=NS),
        scratch_types=[
            pltpu.VMEM_SHARED((N_PAD, HALF), jnp.float32),
            pltpu.VMEM((E_PER_TILE,), jnp.int32),
            pltpu.VMEM((NCHUNK, CHUNK), jnp.int32),
            pltpu.VMEM((CHUNK, HALF), jnp.float32),
            pltpu.VMEM((CHUNK, HALF), jnp.float32),
            pltpu.SemaphoreType.DMA,
            pltpu.SemaphoreType.DMA,
            pltpu.SemaphoreType.DMA,
            pltpu.SemaphoreType.DMA,
            pltpu.SemaphoreType.DMA,
            pltpu.SemaphoreType.DMA,
        ],
    )
    return f(src2d, dst2d, table2)


BN = 1000


def _mlp_body(msg_ref, table_ref, w1_ref, b1_ref, w2_ref, b2_ref, out_ref):
    x0 = msg_ref[0]
    x1 = msg_ref[1]
    h = jnp.dot(x0, w1_ref[:HALF, :], preferred_element_type=jnp.float32)
    h = h + jnp.dot(x1, w1_ref[HALF:, :], preferred_element_type=jnp.float32)
    h = jnp.maximum(h + b1_ref[...], 0.0)
    u = jnp.dot(h, w2_ref[...], preferred_element_type=jnp.float32)
    u = jnp.maximum(u + b2_ref[...], 0.0)
    out_ref[...] = table_ref[...] + u


def _mlp(msg, table, W1, b1, W2, b2):
    return pl.pallas_call(
        _mlp_body,
        grid=(N_NODES // BN,),
        in_specs=[
            pl.BlockSpec((NC, BN, HALF), lambda i: (0, i, 0)),
            pl.BlockSpec((BN, D_EMBED), lambda i: (i, 0)),
            pl.BlockSpec((D_EMBED, D_HIDDEN), lambda i: (0, 0)),
            pl.BlockSpec((1, D_HIDDEN), lambda i: (0, 0)),
            pl.BlockSpec((D_HIDDEN, D_EMBED), lambda i: (0, 0)),
            pl.BlockSpec((1, D_EMBED), lambda i: (0, 0)),
        ],
        out_specs=pl.BlockSpec((BN, D_EMBED), lambda i: (i, 0)),
        out_shape=jax.ShapeDtypeStruct((N_NODES, D_EMBED), jnp.float32),
    )(msg, table, W1, b1, W2, b2)


def kernel(edge_index, table, W1, b1, W2, b2):
    src2d = edge_index[0].reshape(NS, E_PER_TILE)
    dst2d = edge_index[1].reshape(NS, NCHUNK, CHUNK)
    table2 = table.reshape(NC * N_NODES, HALF)
    msg = _sc_messages(src2d, dst2d, table2)
    return _mlp(msg, table, W1, b1.reshape(1, D_HIDDEN), W2, b2.reshape(1, D_EMBED))


# speedup vs baseline: 1.0266x; 1.0034x over previous
import functools

import jax
import jax.numpy as jnp
from jax import lax
from jax.experimental import pallas as pl
from jax.experimental.pallas import tpu as pltpu
from jax.experimental.pallas import tpu_sc as plsc

N_NODES = 10000
D_EMBED = 256
D_HIDDEN = 512
N_EDGES = 160000

NC = 2
NS = 16
LANES = 16
HALF = D_EMBED // NC
CHUNK = 80
E_PER_TILE = N_EDGES // NS
NCHUNK = E_PER_TILE // CHUNK
N_PAD = 10240
ROWS_PER_TILE = N_PAD // NS


def _sc_body(src_hbm, dst_hbm, table_hbm, zeros_hbm, out_hbm,
             acc_sh, src_v, dst_v, data0, data1, gsem0, gsem1, ssem0, ssem1,
             gsem0b, gsem1b):
    c = lax.axis_index("c")
    s = lax.axis_index("s")

    row0 = s * ROWS_PER_TILE
    pltpu.async_copy(src_hbm.at[c, s], src_v, gsem0)
    pltpu.async_copy(dst_hbm.at[s], dst_v, gsem1)
    pltpu.async_copy(zeros_hbm, acc_sh.at[pl.ds(row0, ROWS_PER_TILE)], ssem0)
    pltpu.make_async_copy(src_hbm.at[c, s], src_v, gsem0).wait()
    pltpu.make_async_copy(dst_hbm.at[s], dst_v, gsem1).wait()
    pltpu.make_async_copy(zeros_hbm, acc_sh.at[pl.ds(row0, ROWS_PER_TILE)],
                          ssem0).wait()
    plsc.subcore_barrier()

    bufs = ((data0, gsem0, ssem0), (data1, gsem1, ssem1))

    H = CHUNK // 2
    bsems = (gsem0b, gsem1b)

    def _fire_gather(b, j):
        data, gsem, _ = bufs[b]
        pltpu.async_copy(table_hbm.at[src_v.at[pl.ds(j * CHUNK, H)]],
                         data.at[pl.ds(0, H)], gsem)
        pltpu.async_copy(table_hbm.at[src_v.at[pl.ds(j * CHUNK + H, H)]],
                         data.at[pl.ds(H, H)], bsems[b])

    def _wait_gather(b, j):
        data, gsem, _ = bufs[b]
        pltpu.make_async_copy(table_hbm.at[src_v.at[pl.ds(j * CHUNK, H)]],
                              data.at[pl.ds(0, H)], gsem).wait()
        pltpu.make_async_copy(table_hbm.at[src_v.at[pl.ds(j * CHUNK + H, H)]],
                              data.at[pl.ds(H, H)], bsems[b]).wait()

    def _fire_scatter(b, j):
        data, _, ssem = bufs[b]
        pltpu.async_copy(data, acc_sh.at[dst_v.at[j]], ssem, add=True)

    def _wait_scatter(b, j):
        data, _, ssem = bufs[b]
        pltpu.make_async_copy(data, acc_sh.at[dst_v.at[j]], ssem).wait()

    _fire_gather(0, 0)
    _fire_gather(1, 1)

    @pl.loop(0, NCHUNK, step=2)
    def _mainloop(jj):
        for b in range(2):
            j = jj + b

            @pl.when(j < NCHUNK)
            def _():
                _wait_gather(b, j)
                _fire_scatter(b, j)

                @pl.when(j + 2 < NCHUNK)
                def _():
                    _wait_scatter(b, j)
                    _fire_gather(b, j + 2)

    _wait_scatter(1, NCHUNK - 2)
    _wait_scatter(0, NCHUNK - 1)
    plsc.subcore_barrier()
    pltpu.sync_copy(acc_sh.at[pl.ds(row0, ROWS_PER_TILE)],
                    out_hbm.at[c, pl.ds(row0, ROWS_PER_TILE)])


def _sc_messages(ids, dst2d, table2, zeros):
    f = pl.kernel(
        _sc_body,
        out_type=jax.ShapeDtypeStruct((NC, N_PAD, HALF), jnp.float32),
        mesh=plsc.VectorSubcoreMesh(core_axis_name="c", subcore_axis_name="s",
                                    num_cores=NC, num_subcores=NS),
        scratch_types=[
            pltpu.VMEM_SHARED((N_PAD, HALF), jnp.float32),
            pltpu.VMEM((E_PER_TILE,), jnp.int32),
            pltpu.VMEM((NCHUNK, CHUNK), jnp.int32),
            pltpu.VMEM((CHUNK, HALF), jnp.float32),
            pltpu.VMEM((CHUNK, HALF), jnp.float32),
            pltpu.SemaphoreType.DMA,
            pltpu.SemaphoreType.DMA,
            pltpu.SemaphoreType.DMA,
            pltpu.SemaphoreType.DMA,
            pltpu.SemaphoreType.DMA,
            pltpu.SemaphoreType.DMA,
        ],
    )
    return f(ids, dst2d, table2, zeros)


BN = 1000


def _mlp_body(msg_ref, table_ref, w1_ref, b1_ref, w2_ref, b2_ref, out_ref):
    x0 = msg_ref[0]
    x1 = msg_ref[1]
    h = jnp.dot(x0, w1_ref[:HALF, :], preferred_element_type=jnp.float32)
    h = h + jnp.dot(x1, w1_ref[HALF:, :], preferred_element_type=jnp.float32)
    h = jnp.maximum(h + b1_ref[...], 0.0)
    u = jnp.dot(h, w2_ref[...], preferred_element_type=jnp.float32)
    u = jnp.maximum(u + b2_ref[...], 0.0)
    out_ref[...] = table_ref[...] + u


def _mlp(msg, table, W1, b1, W2, b2):
    return pl.pallas_call(
        _mlp_body,
        grid=(N_NODES // BN,),
        in_specs=[
            pl.BlockSpec((NC, BN, HALF), lambda i: (0, i, 0)),
            pl.BlockSpec((BN, D_EMBED), lambda i: (i, 0)),
            pl.BlockSpec((D_EMBED, D_HIDDEN), lambda i: (0, 0)),
            pl.BlockSpec((1, D_HIDDEN), lambda i: (0, 0)),
            pl.BlockSpec((D_HIDDEN, D_EMBED), lambda i: (0, 0)),
            pl.BlockSpec((1, D_EMBED), lambda i: (0, 0)),
        ],
        out_specs=pl.BlockSpec((BN, D_EMBED), lambda i: (i, 0)),
        out_shape=jax.ShapeDtypeStruct((N_NODES, D_EMBED), jnp.float32),
    )(msg, table, W1, b1, W2, b2)


def kernel(edge_index, table, W1, b1, W2, b2):
    ids = ((edge_index[0] * 2).reshape(1, NS, E_PER_TILE)
           + jnp.arange(NC, dtype=jnp.int32).reshape(NC, 1, 1))
    dst2d = edge_index[1].reshape(NS, NCHUNK, CHUNK)
    table2 = table.reshape(NC * N_NODES, HALF)
    zeros = jnp.zeros((ROWS_PER_TILE, HALF), jnp.float32)
    msg = _sc_messages(ids, dst2d, table2, zeros)
    return _mlp(msg, table, W1, b1.reshape(1, D_HIDDEN), W2, b2.reshape(1, D_EMBED))


# speedup vs baseline: 1.0408x; 1.0138x over previous
import functools

import jax
import jax.numpy as jnp
from jax import lax
from jax.experimental import pallas as pl
from jax.experimental.pallas import tpu as pltpu
from jax.experimental.pallas import tpu_sc as plsc

N_NODES = 10000
D_EMBED = 256
D_HIDDEN = 512
N_EDGES = 160000

NC = 2
NS = 16
LANES = 16
HALF = D_EMBED // NC
CHUNK = 80
E_PER_TILE = N_EDGES // NS
NCHUNK = E_PER_TILE // CHUNK
N_PAD = 10240
ROWS_PER_TILE = N_PAD // NS


def _sc_body(src_hbm, dst_hbm, table_hbm, zeros_hbm, out_hbm,
             acc_sh, src_v, dst_v, data0, data1, gsem0, gsem1, ssem0, ssem1,
             gsem0b, gsem1b):
    c = lax.axis_index("c")
    s = lax.axis_index("s")

    row0 = s * ROWS_PER_TILE
    pltpu.async_copy(src_hbm.at[c, s], src_v, gsem0)
    pltpu.async_copy(dst_hbm.at[s], dst_v, gsem1)
    pltpu.async_copy(zeros_hbm, acc_sh.at[pl.ds(row0, ROWS_PER_TILE)], ssem0)
    pltpu.make_async_copy(src_hbm.at[c, s], src_v, gsem0).wait()
    pltpu.make_async_copy(dst_hbm.at[s], dst_v, gsem1).wait()
    pltpu.make_async_copy(zeros_hbm, acc_sh.at[pl.ds(row0, ROWS_PER_TILE)],
                          ssem0).wait()
    plsc.subcore_barrier()

    bufs = ((data0, gsem0, ssem0), (data1, gsem1, ssem1))

    H = CHUNK // 2
    bsems = (gsem0b, gsem1b)

    def _fire_gather(b, j):
        data, gsem, _ = bufs[b]
        pltpu.async_copy(table_hbm.at[src_v.at[pl.ds(j * CHUNK, H)]],
                         data.at[pl.ds(0, H)], gsem)
        pltpu.async_copy(table_hbm.at[src_v.at[pl.ds(j * CHUNK + H, H)]],
                         data.at[pl.ds(H, H)], bsems[b])

    def _wait_gather(b, j):
        data, gsem, _ = bufs[b]
        pltpu.make_async_copy(table_hbm.at[src_v.at[pl.ds(j * CHUNK, H)]],
                              data.at[pl.ds(0, H)], gsem).wait()
        pltpu.make_async_copy(table_hbm.at[src_v.at[pl.ds(j * CHUNK + H, H)]],
                              data.at[pl.ds(H, H)], bsems[b]).wait()

    def _fire_scatter(b, j):
        data, _, ssem = bufs[b]
        pltpu.async_copy(data, acc_sh.at[dst_v.at[j]], ssem, add=True)

    def _wait_scatter(b, j):
        data, _, ssem = bufs[b]
        pltpu.make_async_copy(data, acc_sh.at[dst_v.at[j]], ssem).wait()

    _fire_gather(0, 0)
    _fire_gather(1, 1)

    @pl.loop(0, NCHUNK, step=2)
    def _mainloop(jj):
        for b in range(2):
            j = jj + b

            @pl.when(j < NCHUNK)
            def _():
                _wait_gather(b, j)
                _fire_scatter(b, j)

                @pl.when(j + 2 < NCHUNK)
                def _():
                    _wait_scatter(b, j)
                    _fire_gather(b, j + 2)

    _wait_scatter(1, NCHUNK - 2)
    _wait_scatter(0, NCHUNK - 1)
    plsc.subcore_barrier()
    pltpu.sync_copy(acc_sh.at[pl.ds(row0, ROWS_PER_TILE)],
                    out_hbm.at[c, pl.ds(row0, ROWS_PER_TILE)])


def _sc_messages(ids, dst2d, table2, zeros):
    f = pl.kernel(
        _sc_body,
        out_type=jax.ShapeDtypeStruct((NC, N_PAD, HALF), jnp.float32),
        mesh=plsc.VectorSubcoreMesh(core_axis_name="c", subcore_axis_name="s",
                                    num_cores=NC, num_subcores=NS),
        scratch_types=[
            pltpu.VMEM_SHARED((N_PAD, HALF), jnp.float32),
            pltpu.VMEM((E_PER_TILE,), jnp.int32),
            pltpu.VMEM((NCHUNK, CHUNK), jnp.int32),
            pltpu.VMEM((CHUNK, HALF), jnp.float32),
            pltpu.VMEM((CHUNK, HALF), jnp.float32),
            pltpu.SemaphoreType.DMA,
            pltpu.SemaphoreType.DMA,
            pltpu.SemaphoreType.DMA,
            pltpu.SemaphoreType.DMA,
            pltpu.SemaphoreType.DMA,
            pltpu.SemaphoreType.DMA,
        ],
    )
    return f(ids, dst2d, table2, zeros)


BN = 2000


def _mlp_body(msg_ref, table_ref, w1_ref, b1_ref, w2_ref, b2_ref, out_ref):
    x0 = msg_ref[0]
    x1 = msg_ref[1]
    h = jnp.dot(x0, w1_ref[:HALF, :], preferred_element_type=jnp.float32)
    h = h + jnp.dot(x1, w1_ref[HALF:, :], preferred_element_type=jnp.float32)
    h = jnp.maximum(h + b1_ref[...], 0.0)
    u = jnp.dot(h, w2_ref[...], preferred_element_type=jnp.float32)
    u = jnp.maximum(u + b2_ref[...], 0.0)
    out_ref[...] = table_ref[...] + u


def _mlp(msg, table, W1, b1, W2, b2):
    return pl.pallas_call(
        _mlp_body,
        grid=(N_NODES // BN,),
        in_specs=[
            pl.BlockSpec((NC, BN, HALF), lambda i: (0, i, 0)),
            pl.BlockSpec((BN, D_EMBED), lambda i: (i, 0)),
            pl.BlockSpec((D_EMBED, D_HIDDEN), lambda i: (0, 0)),
            pl.BlockSpec((1, D_HIDDEN), lambda i: (0, 0)),
            pl.BlockSpec((D_HIDDEN, D_EMBED), lambda i: (0, 0)),
            pl.BlockSpec((1, D_EMBED), lambda i: (0, 0)),
        ],
        out_specs=pl.BlockSpec((BN, D_EMBED), lambda i: (i, 0)),
        out_shape=jax.ShapeDtypeStruct((N_NODES, D_EMBED), jnp.float32),
    )(msg, table, W1, b1, W2, b2)


def kernel(edge_index, table, W1, b1, W2, b2):
    ids = ((edge_index[0] * 2).reshape(1, NS, E_PER_TILE)
           + jnp.arange(NC, dtype=jnp.int32).reshape(NC, 1, 1))
    dst2d = edge_index[1].reshape(NS, NCHUNK, CHUNK)
    table2 = table.reshape(NC * N_NODES, HALF)
    zeros = jnp.zeros((ROWS_PER_TILE, HALF), jnp.float32)
    msg = _sc_messages(ids, dst2d, table2, zeros)
    return _mlp(msg, table, W1, b1.reshape(1, D_HIDDEN), W2, b2.reshape(1, D_EMBED))


# speedup vs baseline: 1.0421x; 1.0012x over previous
import functools

import jax
import jax.numpy as jnp
from jax import lax
from jax.experimental import pallas as pl
from jax.experimental.pallas import tpu as pltpu
from jax.experimental.pallas import tpu_sc as plsc

N_NODES = 10000
D_EMBED = 256
D_HIDDEN = 512
N_EDGES = 160000

NC = 2
NS = 16
LANES = 16
HALF = D_EMBED // NC
CHUNK = 80
E_PER_TILE = N_EDGES // NS
NCHUNK = E_PER_TILE // CHUNK
N_PAD = 10240
ROWS_PER_TILE = N_PAD // NS


def _sc_body(src_hbm, dst_hbm, table_hbm, zeros_hbm, out_hbm,
             acc_sh, src_v, dst_v, data0, data1, gsem0, gsem1, ssem0, ssem1,
             gsem0b, gsem1b):
    c = lax.axis_index("c")
    s = lax.axis_index("s")

    row0 = s * ROWS_PER_TILE
    pltpu.async_copy(src_hbm.at[c, s], src_v, gsem0)
    pltpu.async_copy(dst_hbm.at[s], dst_v, gsem1)
    pltpu.async_copy(zeros_hbm, acc_sh.at[pl.ds(row0, ROWS_PER_TILE)], ssem0)
    pltpu.make_async_copy(src_hbm.at[c, s], src_v, gsem0).wait()
    pltpu.make_async_copy(dst_hbm.at[s], dst_v, gsem1).wait()
    pltpu.make_async_copy(zeros_hbm, acc_sh.at[pl.ds(row0, ROWS_PER_TILE)],
                          ssem0).wait()
    plsc.subcore_barrier()

    bufs = ((data0, gsem0, ssem0), (data1, gsem1, ssem1))

    H = CHUNK // 2
    bsems = (gsem0b, gsem1b)

    def _fire_gather(b, j):
        data, gsem, _ = bufs[b]
        pltpu.async_copy(table_hbm.at[src_v.at[pl.ds(j * CHUNK, H)]],
                         data.at[pl.ds(0, H)], gsem)
        pltpu.async_copy(table_hbm.at[src_v.at[pl.ds(j * CHUNK + H, H)]],
                         data.at[pl.ds(H, H)], bsems[b])

    def _wait_gather(b, j):
        data, gsem, _ = bufs[b]
        pltpu.make_async_copy(table_hbm.at[src_v.at[pl.ds(j * CHUNK, H)]],
                              data.at[pl.ds(0, H)], gsem).wait()
        pltpu.make_async_copy(table_hbm.at[src_v.at[pl.ds(j * CHUNK + H, H)]],
                              data.at[pl.ds(H, H)], bsems[b]).wait()

    def _fire_scatter(b, j):
        data, _, ssem = bufs[b]
        pltpu.async_copy(data, acc_sh.at[dst_v.at[j]], ssem, add=True)

    def _wait_scatter(b, j):
        data, _, ssem = bufs[b]
        pltpu.make_async_copy(data, acc_sh.at[dst_v.at[j]], ssem).wait()

    _fire_gather(0, 0)
    _fire_gather(1, 1)

    @pl.loop(0, NCHUNK, step=2)
    def _mainloop(jj):
        for b in range(2):
            j = jj + b

            @pl.when(j < NCHUNK)
            def _():
                _wait_gather(b, j)
                _fire_scatter(b, j)

                @pl.when(j + 2 < NCHUNK)
                def _():
                    _wait_scatter(b, j)
                    _fire_gather(b, j + 2)

    _wait_scatter(1, NCHUNK - 2)
    _wait_scatter(0, NCHUNK - 1)
    plsc.subcore_barrier()
    pltpu.sync_copy(acc_sh.at[pl.ds(row0, ROWS_PER_TILE)],
                    out_hbm.at[c, pl.ds(row0, ROWS_PER_TILE)])


def _sc_messages(ids, dst2d, table2, zeros):
    f = pl.kernel(
        _sc_body,
        out_type=jax.ShapeDtypeStruct((NC, N_PAD, HALF), jnp.float32),
        mesh=plsc.VectorSubcoreMesh(core_axis_name="c", subcore_axis_name="s",
                                    num_cores=NC, num_subcores=NS),
        scratch_types=[
            pltpu.VMEM_SHARED((N_PAD, HALF), jnp.float32),
            pltpu.VMEM((E_PER_TILE,), jnp.int32),
            pltpu.VMEM((NCHUNK, CHUNK), jnp.int32),
            pltpu.VMEM((CHUNK, HALF), jnp.float32),
            pltpu.VMEM((CHUNK, HALF), jnp.float32),
            pltpu.SemaphoreType.DMA,
            pltpu.SemaphoreType.DMA,
            pltpu.SemaphoreType.DMA,
            pltpu.SemaphoreType.DMA,
            pltpu.SemaphoreType.DMA,
            pltpu.SemaphoreType.DMA,
        ],
    )
    return f(ids, dst2d, table2, zeros)


BN = 5000


def _mlp_body(msg_ref, table_ref, w1_ref, b1_ref, w2_ref, b2_ref, out_ref):
    x0 = msg_ref[0]
    x1 = msg_ref[1]
    h = jnp.dot(x0, w1_ref[:HALF, :], preferred_element_type=jnp.float32)
    h = h + jnp.dot(x1, w1_ref[HALF:, :], preferred_element_type=jnp.float32)
    h = jnp.maximum(h + b1_ref[...], 0.0)
    u = jnp.dot(h, w2_ref[...], preferred_element_type=jnp.float32)
    u = jnp.maximum(u + b2_ref[...], 0.0)
    out_ref[...] = table_ref[...] + u


def _mlp(msg, table, W1, b1, W2, b2):
    return pl.pallas_call(
        _mlp_body,
        grid=(N_NODES // BN,),
        in_specs=[
            pl.BlockSpec((NC, BN, HALF), lambda i: (0, i, 0)),
            pl.BlockSpec((BN, D_EMBED), lambda i: (i, 0)),
            pl.BlockSpec((D_EMBED, D_HIDDEN), lambda i: (0, 0)),
            pl.BlockSpec((1, D_HIDDEN), lambda i: (0, 0)),
            pl.BlockSpec((D_HIDDEN, D_EMBED), lambda i: (0, 0)),
            pl.BlockSpec((1, D_EMBED), lambda i: (0, 0)),
        ],
        out_specs=pl.BlockSpec((BN, D_EMBED), lambda i: (i, 0)),
        out_shape=jax.ShapeDtypeStruct((N_NODES, D_EMBED), jnp.float32),
    )(msg, table, W1, b1, W2, b2)


def kernel(edge_index, table, W1, b1, W2, b2):
    ids = ((edge_index[0] * 2).reshape(1, NS, E_PER_TILE)
           + jnp.arange(NC, dtype=jnp.int32).reshape(NC, 1, 1))
    dst2d = edge_index[1].reshape(NS, NCHUNK, CHUNK)
    table2 = table.reshape(NC * N_NODES, HALF)
    zeros = jnp.zeros((ROWS_PER_TILE, HALF), jnp.float32)
    msg = _sc_messages(ids, dst2d, table2, zeros)
    return _mlp(msg, table, W1, b1.reshape(1, D_HIDDEN), W2, b2.reshape(1, D_EMBED))
